# baseline (speedup 1.0000x reference)
import functools

import jax
import jax.numpy as jnp
import numpy as np
from jax import lax
from jax.experimental import pallas as pl
from jax.experimental.pallas import tpu as pltpu
from jax.experimental.pallas import tpu_sc as plsc

B = 16384
F = 26
NC, NS, L = 2, 16, 16
NW = NC * NS
BW = B // NW
NIDX = F * BW
CHUNK = 128
NCHUNK = NIDX // CHUNK
PAT = 208
NPH = PAT // L

_ROWID = np.arange(NIDX, dtype=np.int32) // F


def _sc_body(x_hbm, fc_hbm, offp_hbm, biasb_hbm, rowid_hbm, out_hbm,
             xv, rows, outv, offv, biasv, ridv, sem):
    wid = lax.axis_index("s") * NC + lax.axis_index("c")
    base = wid * NIDX

    pltpu.sync_copy(x_hbm.at[pl.ds(0, 1), pl.ds(base, NIDX)], xv)
    pltpu.sync_copy(offp_hbm, offv)
    pltpu.sync_copy(biasb_hbm, biasv)
    pltpu.sync_copy(rowid_hbm, ridv)

    for j in range(NCHUNK):
        for k in range(CHUNK // L):
            ph = ((j * (CHUNK // L)) + k) % NPH
            s = j * CHUNK + k * L
            xv[0, pl.ds(s, L)] = xv[0, pl.ds(s, L)] + offv[pl.ds(ph * L, L)]
        pltpu.make_async_copy(
            fc_hbm.at[xv.at[pl.ds(0, 1), pl.ds(j * CHUNK, CHUNK)]],
            rows.at[pl.ds(0, 1), pl.ds(j * CHUNK, CHUNK)],
            sem,
        ).start()

    bias_vec = biasv[:]
    @pl.loop(0, BW // L)
    def _init(c):
        outv[pl.ds(c * L, L)] = bias_vec

    pltpu.make_async_copy(
        fc_hbm.at[pl.ds(0, 1), pl.ds(0, NIDX)], rows, sem
    ).wait()

    @pl.loop(0, NIDX // L)
    def _reduce(c):
        vals = rows[0, pl.ds(c * L, L)]
        rid = ridv[pl.ds(c * L, L)]
        plsc.addupdate_scatter(outv, [rid], vals)

    pltpu.sync_copy(outv, out_hbm.at[pl.ds(wid * BW, BW)])


@jax.jit
def _features_linear(xr, fcr, offp, biasb, rowid):
    mesh = plsc.VectorSubcoreMesh(core_axis_name="c", subcore_axis_name="s")
    return pl.kernel(
        _sc_body,
        out_type=jax.ShapeDtypeStruct((B,), jnp.float32),
        mesh=mesh,
        compiler_params=pltpu.CompilerParams(needs_layout_passes=False),
        scratch_types=[
            pltpu.VMEM((1, NIDX), jnp.int32),
            pltpu.VMEM((1, NIDX), jnp.float32),
            pltpu.VMEM((BW,), jnp.float32),
            pltpu.VMEM((PAT,), jnp.int32),
            pltpu.VMEM((L,), jnp.float32),
            pltpu.VMEM((NIDX,), jnp.int32),
            pltpu.SemaphoreType.DMA,
        ],
    )(xr, fcr, offp, biasb, rowid)


def kernel(x, fc, bias, offsets):
    xr = x.reshape(1, B * F)
    fcr = fc.reshape(1, -1)
    offp = jnp.tile(offsets, PAT // F)
    biasb = jnp.broadcast_to(bias, (L,))
    rowid = jnp.asarray(_ROWID)
    out = _features_linear(xr, fcr, offp, biasb, rowid)
    return out.reshape(B, 1)


# speedup vs baseline: 1.8285x; 1.8285x over previous
import functools

import jax
import jax.numpy as jnp
from jax import lax
from jax.experimental import pallas as pl
from jax.experimental.pallas import tpu as pltpu
from jax.experimental.pallas import tpu_sc as plsc

B = 16384
F = 26
NC, NS, L = 2, 16, 16
NW = NC * NS
BW = B // NW
NIDX = F * BW
CHUNK = 128
NCH = BW // CHUNK


def _sc_body(xt_hbm, fc_hbm, offb_hbm, biasb_hbm, out_hbm,
             xv, rows, outv, offv, biasv, sem):
    wid = lax.axis_index("s") * NC + lax.axis_index("c")
    base = wid * BW

    pltpu.sync_copy(xt_hbm.at[:, pl.ds(base, BW)], xv)
    pltpu.sync_copy(offb_hbm, offv)
    pltpu.sync_copy(biasb_hbm, biasv)

    for f in range(F):
        off_f = offv[f, :]
        for c4 in range(NCH):
            @pl.loop(c4 * (CHUNK // L), (c4 + 1) * (CHUNK // L))
            def _idx(c):
                xv[f, pl.ds(c * L, L)] = xv[f, pl.ds(c * L, L)] + off_f
            pltpu.make_async_copy(
                fc_hbm.at[xv.at[pl.ds(f, 1), pl.ds(c4 * CHUNK, CHUNK)]],
                rows.at[pl.ds(0, 1), pl.ds(f * BW + c4 * CHUNK, CHUNK)],
                sem,
            ).start()

    pltpu.make_async_copy(
        fc_hbm.at[pl.ds(0, 1), pl.ds(0, NIDX)], rows, sem
    ).wait()

    bias_vec = biasv[:]
    @pl.loop(0, BW // L)
    def _reduce(c):
        acc = bias_vec
        for f in range(F):
            acc = acc + rows[0, pl.ds(f * BW + c * L, L)]
        outv[pl.ds(c * L, L)] = acc

    pltpu.sync_copy(outv, out_hbm.at[pl.ds(base, BW)])


@jax.jit
def _features_linear(xt, fcr, offb, biasb):
    mesh = plsc.VectorSubcoreMesh(core_axis_name="c", subcore_axis_name="s")
    return pl.kernel(
        _sc_body,
        out_type=jax.ShapeDtypeStruct((B,), jnp.float32),
        mesh=mesh,
        scratch_types=[
            pltpu.VMEM((F, BW), jnp.int32),
            pltpu.VMEM((1, NIDX), jnp.float32),
            pltpu.VMEM((BW,), jnp.float32),
            pltpu.VMEM((F, L), jnp.int32),
            pltpu.VMEM((L,), jnp.float32),
            pltpu.SemaphoreType.DMA,
        ],
    )(xt, fcr, offb, biasb)


def kernel(x, fc, bias, offsets):
    xt = x.T
    offb = jnp.broadcast_to(offsets[:, None], (F, L))
    biasb = jnp.broadcast_to(bias, (L,))
    out = _features_linear(xt, fc.reshape(1, -1), offb, biasb)
    return out.reshape(B, 1)
